# baseline (speedup 1.0000x reference)
import jax
import jax.numpy as jnp
from jax import lax
from jax.experimental import pallas as pl
from jax.experimental.pallas import tpu as pltpu

_PW = 2.0
_ALPHA = 0.75
_HNM = 10
_TVA = 0.3
_TVB = 0.7


def _rsum(x):
    return jnp.sum(jnp.sum(x, axis=0))


def _msum(x):
    ones = jnp.ones((1, x.shape[0]), jnp.float32)
    cs = jnp.dot(ones, x, precision=lax.Precision.HIGHEST,
                 preferred_element_type=jnp.float32)
    return jnp.sum(cs)


def _focal_map(praw, t):
    p = jnp.clip(praw, 1e-6, 1 - 1e-6)
    is_pos = t == 1.0
    p_t = jnp.where(is_pos, p, 1.0 - p)
    q = 1.0 - p_t
    coef = jnp.where(is_pos, _ALPHA * _PW, 1.0 - _ALPHA)
    focal = -(coef * q * q) * jnp.log(p_t)
    return p, is_pos, focal


def _body(prob_ref, label_ref, rlog_ref, rsp_ref, pp_ref, pt_ref, mu_ref,
          std_ref, out_ref):
    b = pl.program_id(0)
    praw = prob_ref[0]
    t = label_ref[0]

    p, _, focal = _focal_map(praw, t)
    n_pos_f = _msum(t)
    sum_pos = _msum(focal * t)
    neg_all = _msum(focal) - sum_pos

    tp = _msum(p * t)
    fp = _msum(p) - tp
    fn = n_pos_f - tp
    tv_b = 1.0 - (tp + 1.0) / (tp + _TVA * fp + _TVB * fn + 1.0)

    r = rsp_ref[0]
    rlt = jnp.log(1.0 + jnp.maximum(r, 0.0))
    gate = jnp.logical_or(praw > 0.1, r > 1.0).astype(jnp.float32)
    heavy = (r >= 50.0).astype(jnp.float32)
    w = gate * (1.0 + 3.0 * heavy)
    d = rlog_ref[0] - rlt
    ad = jnp.abs(d)
    hub = jnp.where(ad < 1.0, 0.5 * d * d, ad - 0.5)

    out_ref[0, 0, 0] = sum_pos
    out_ref[0, 0, 1] = tv_b
    out_ref[0, 0, 2] = _msum(hub * w)
    out_ref[0, 0, 3] = _msum(w)
    out_ref[0, 0, 5] = n_pos_f
    out_ref[0, 0, 6] = neg_all

    @pl.when(b == 0)
    def _aux():
        norm = (pt_ref[...] - mu_ref[...]) / (std_ref[...] + 1e-6)
        norm = jnp.where(jnp.isnan(norm), 0.0, norm)
        out_ref[0, 0, 4] = jnp.mean((pp_ref[...] - norm) ** 2)

    @pl.when(b != 0)
    def _aux0():
        out_ref[0, 0, 4] = 0.0


def _fb_body(prob_ref, label_ref, out_ref):
    praw = prob_ref[0]
    t = label_ref[0]
    H, W = praw.shape
    N = H * W
    _, is_pos, focal = _focal_map(praw, t)
    n_pos_i = _rsum(t).astype(jnp.int32)
    n_neg_i = N - n_pos_i
    k = jnp.minimum(n_pos_i * _HNM, n_neg_i)

    vals = jnp.where(is_pos, -1.0, focal)
    vbits = lax.bitcast_convert_type(vals, jnp.int32)

    def step(_, lh):
        lo, hi = lh
        mid = lo + (hi - lo + 1) // 2
        cnt = jnp.sum(jnp.sum((vbits >= mid).astype(jnp.int32), axis=0))
        take = cnt >= k
        return (jnp.where(take, mid, lo), jnp.where(take, hi, mid - 1))

    lo, _ = lax.fori_loop(0, 31, step, (jnp.int32(0), jnp.int32(0x7F7FFFFF)))
    gt = vbits > lo
    eq = vbits == lo
    cnt_gt = _rsum(gt.astype(jnp.float32))
    cnt_eq = jnp.maximum(_rsum(eq.astype(jnp.float32)), 1.0)
    sum_gt = _rsum(jnp.where(gt, focal, 0.0))
    sum_eq = _rsum(jnp.where(eq, focal, 0.0))
    out_ref[0, 0, 0] = (sum_gt
                        + (k.astype(jnp.float32) - cnt_gt) * sum_eq / cnt_eq)


def kernel(prob_map, rain_logit, pred_phys, label_map, rain_max_true,
           rain_spatial_true, phys_targets, phys_mu, phys_std):
    B, H, W = prob_map.shape
    N = H * W
    P = pred_phys.shape[1]
    mu_b = jnp.broadcast_to(phys_mu[None, :], (B, P))
    std_b = jnp.broadcast_to(phys_std[None, :], (B, P))

    img = pl.BlockSpec((1, H, W), lambda b: (b, 0, 0))
    small = pl.BlockSpec((B, P), lambda b: (0, 0))
    stats = pl.pallas_call(
        _body,
        grid=(B,),
        in_specs=[img, img, img, img, small, small, small, small],
        out_specs=pl.BlockSpec((1, 1, 8), lambda b: (b, 0, 0),
                               memory_space=pltpu.SMEM),
        out_shape=jax.ShapeDtypeStruct((B, 1, 8), jnp.float32),
    )(prob_map, label_map, rain_logit, rain_spatial_true,
      pred_phys, phys_targets, mu_b, std_b)

    stats = stats[:, 0, :]
    sum_pos = stats[:, 0]
    tv_b = stats[:, 1]
    n_pos_f = stats[:, 5]
    neg_all = stats[:, 6]

    n_pos_i = n_pos_f.astype(jnp.int32)
    n_neg_i = N - n_pos_i
    n_hard_i = jnp.minimum(n_pos_i * _HNM, n_neg_i)
    common = n_hard_i == n_neg_i

    def _fallback():
        fb = pl.pallas_call(
            _fb_body,
            grid=(B,),
            in_specs=[img, img],
            out_specs=pl.BlockSpec((1, 1, 8), lambda b: (b, 0, 0),
                                   memory_space=pltpu.SMEM),
            out_shape=jax.ShapeDtypeStruct((B, 1, 8), jnp.float32),
        )(prob_map, label_map)
        return jnp.where(common, neg_all, fb[:, 0, 0])

    sum_hard = lax.cond(jnp.all(common), lambda: neg_all, _fallback)

    fl = jnp.mean((sum_pos + sum_hard)
                  / (n_pos_f + n_hard_i.astype(jnp.float32)))
    tv = jnp.mean(tv_b)
    reg = jnp.sum(stats[:, 2]) / jnp.maximum(jnp.sum(stats[:, 3]), 1.0)
    aux = stats[0, 4]
    total = fl + 0.5 * tv + 1.0 * reg + 0.1 * aux
    return (total, fl, tv, reg, aux)


# speedup vs baseline: 1.3451x; 1.3451x over previous
import jax
import jax.numpy as jnp
from jax import lax
from jax.experimental import pallas as pl
from jax.experimental.pallas import tpu as pltpu

_PW = 2.0
_ALPHA = 0.75
_HNM = 10
_TVA = 0.3
_TVB = 0.7


def _rsum(x):
    return jnp.sum(jnp.sum(x, axis=0))


def _focal_map(praw, t):
    p = jnp.clip(praw, 1e-6, 1 - 1e-6)
    is_pos = t == 1.0
    p_t = jnp.where(is_pos, p, 1.0 - p)
    q = 1.0 - p_t
    coef = jnp.where(is_pos, _ALPHA * _PW, 1.0 - _ALPHA)
    focal = -(coef * q * q) * jnp.log(p_t)
    return p, is_pos, focal


def _body(prob_ref, label_ref, rlog_ref, rsp_ref, pp_ref, pt_ref, mu_ref,
          std_ref, out_ref):
    b = pl.program_id(0)
    praw = prob_ref[0]
    t = label_ref[0]

    p, _, focal = _focal_map(praw, t)
    n_pos_f = _rsum(t)
    sum_pos = _rsum(focal * t)
    neg_all = _rsum(focal) - sum_pos

    tp = _rsum(p * t)
    fp = _rsum(p) - tp
    fn = n_pos_f - tp
    tv_b = 1.0 - (tp + 1.0) / (tp + _TVA * fp + _TVB * fn + 1.0)

    r = rsp_ref[0]
    rlt = jnp.log(1.0 + jnp.maximum(r, 0.0))
    gate = jnp.logical_or(praw > 0.1, r > 1.0).astype(jnp.float32)
    heavy = (r >= 50.0).astype(jnp.float32)
    w = gate * (1.0 + 3.0 * heavy)
    d = rlog_ref[0] - rlt
    ad = jnp.abs(d)
    hub = jnp.where(ad < 1.0, 0.5 * d * d, ad - 0.5)

    out_ref[0, 0, 0] = sum_pos
    out_ref[0, 0, 1] = tv_b
    out_ref[0, 0, 2] = _rsum(hub * w)
    out_ref[0, 0, 3] = _rsum(w)
    out_ref[0, 0, 5] = n_pos_f
    out_ref[0, 0, 6] = neg_all

    @pl.when(b == 0)
    def _aux():
        norm = (pt_ref[...] - mu_ref[...]) / (std_ref[...] + 1e-6)
        norm = jnp.where(jnp.isnan(norm), 0.0, norm)
        out_ref[0, 0, 4] = jnp.mean((pp_ref[...] - norm) ** 2)

    @pl.when(b != 0)
    def _aux0():
        out_ref[0, 0, 4] = 0.0


def _fb_body(prob_ref, label_ref, out_ref):
    praw = prob_ref[0]
    t = label_ref[0]
    H, W = praw.shape
    N = H * W
    _, is_pos, focal = _focal_map(praw, t)
    n_pos_i = _rsum(t).astype(jnp.int32)
    n_neg_i = N - n_pos_i
    k = jnp.minimum(n_pos_i * _HNM, n_neg_i)

    vals = jnp.where(is_pos, -1.0, focal)
    vbits = lax.bitcast_convert_type(vals, jnp.int32)

    def step(_, lh):
        lo, hi = lh
        mid = lo + (hi - lo + 1) // 2
        cnt = jnp.sum(jnp.sum((vbits >= mid).astype(jnp.int32), axis=0))
        take = cnt >= k
        return (jnp.where(take, mid, lo), jnp.where(take, hi, mid - 1))

    lo, _ = lax.fori_loop(0, 31, step, (jnp.int32(0), jnp.int32(0x7F7FFFFF)))
    gt = vbits > lo
    eq = vbits == lo
    cnt_gt = _rsum(gt.astype(jnp.float32))
    cnt_eq = jnp.maximum(_rsum(eq.astype(jnp.float32)), 1.0)
    sum_gt = _rsum(jnp.where(gt, focal, 0.0))
    sum_eq = _rsum(jnp.where(eq, focal, 0.0))
    out_ref[0, 0, 0] = (sum_gt
                        + (k.astype(jnp.float32) - cnt_gt) * sum_eq / cnt_eq)


def kernel(prob_map, rain_logit, pred_phys, label_map, rain_max_true,
           rain_spatial_true, phys_targets, phys_mu, phys_std):
    B, H, W = prob_map.shape
    N = H * W
    P = pred_phys.shape[1]
    mu_b = jnp.broadcast_to(phys_mu[None, :], (B, P))
    std_b = jnp.broadcast_to(phys_std[None, :], (B, P))

    img = pl.BlockSpec((1, H, W), lambda b: (b, 0, 0))
    small = pl.BlockSpec((B, P), lambda b: (0, 0))
    stats = pl.pallas_call(
        _body,
        grid=(B,),
        in_specs=[img, img, img, img, small, small, small, small],
        out_specs=pl.BlockSpec((1, 1, 8), lambda b: (b, 0, 0),
                               memory_space=pltpu.SMEM),
        out_shape=jax.ShapeDtypeStruct((B, 1, 8), jnp.float32),
        compiler_params=pltpu.CompilerParams(
            dimension_semantics=("parallel",)),
    )(prob_map, label_map, rain_logit, rain_spatial_true,
      pred_phys, phys_targets, mu_b, std_b)

    stats = stats[:, 0, :]
    sum_pos = stats[:, 0]
    tv_b = stats[:, 1]
    n_pos_f = stats[:, 5]
    neg_all = stats[:, 6]

    n_pos_i = n_pos_f.astype(jnp.int32)
    n_neg_i = N - n_pos_i
    n_hard_i = jnp.minimum(n_pos_i * _HNM, n_neg_i)
    common = n_hard_i == n_neg_i

    def _fallback():
        fb = pl.pallas_call(
            _fb_body,
            grid=(B,),
            in_specs=[img, img],
            out_specs=pl.BlockSpec((1, 1, 8), lambda b: (b, 0, 0),
                                   memory_space=pltpu.SMEM),
            out_shape=jax.ShapeDtypeStruct((B, 1, 8), jnp.float32),
        )(prob_map, label_map)
        return jnp.where(common, neg_all, fb[:, 0, 0])

    sum_hard = lax.cond(jnp.all(common), lambda: neg_all, _fallback)

    fl = jnp.mean((sum_pos + sum_hard)
                  / (n_pos_f + n_hard_i.astype(jnp.float32)))
    tv = jnp.mean(tv_b)
    reg = jnp.sum(stats[:, 2]) / jnp.maximum(jnp.sum(stats[:, 3]), 1.0)
    aux = stats[0, 4]
    total = fl + 0.5 * tv + 1.0 * reg + 0.1 * aux
    return (total, fl, tv, reg, aux)


# speedup vs baseline: 1.4706x; 1.0933x over previous
import jax
import jax.numpy as jnp
from jax import lax
from jax.experimental import pallas as pl
from jax.experimental.pallas import tpu as pltpu

_PW = 2.0
_ALPHA = 0.75
_HNM = 10
_TVA = 0.3
_TVB = 0.7


def _rsum(x):
    return jnp.sum(jnp.sum(x, axis=0))


def _focal_map(praw, t):
    p = jnp.clip(praw, 1e-6, 1 - 1e-6)
    is_pos = t == 1.0
    p_t = jnp.where(is_pos, p, 1.0 - p)
    q = 1.0 - p_t
    coef = jnp.where(is_pos, _ALPHA * _PW, 1.0 - _ALPHA)
    focal = -(coef * q * q) * jnp.log(p_t)
    return p, is_pos, focal


def _body(prob_ref, label_ref, rlog_ref, rsp_ref, pp_ref, pt_ref, mu_ref,
          std_ref, out_ref):
    b = pl.program_id(0)
    praw = prob_ref[0]
    t = label_ref[0]

    p, _, focal = _focal_map(praw, t)
    n_pos_f = _rsum(t)
    sum_focal = _rsum(focal)

    tp = _rsum(p * t)
    fp = _rsum(p) - tp
    fn = n_pos_f - tp
    tv_b = 1.0 - (tp + 1.0) / (tp + _TVA * fp + _TVB * fn + 1.0)

    r = rsp_ref[0]
    rlt = jnp.log(1.0 + jnp.maximum(r, 0.0))
    gate = jnp.logical_or(praw > 0.1, r > 1.0).astype(jnp.float32)
    heavy = (r >= 50.0).astype(jnp.float32)
    w = gate + 3.0 * heavy
    d = rlog_ref[0] - rlt
    ad = jnp.abs(d)
    hub = jnp.where(ad < 1.0, 0.5 * d * d, ad - 0.5)

    out_ref[0, 0, 0] = sum_focal
    out_ref[0, 0, 1] = tv_b
    out_ref[0, 0, 2] = _rsum(hub * w)
    out_ref[0, 0, 3] = _rsum(w)
    out_ref[0, 0, 5] = n_pos_f

    @pl.when(b == 0)
    def _aux():
        norm = (pt_ref[...] - mu_ref[...]) / (std_ref[...] + 1e-6)
        norm = jnp.where(jnp.isnan(norm), 0.0, norm)
        out_ref[0, 0, 4] = jnp.mean((pp_ref[...] - norm) ** 2)

    @pl.when(b != 0)
    def _aux0():
        out_ref[0, 0, 4] = 0.0


def _fb_body(prob_ref, label_ref, out_ref):
    praw = prob_ref[0]
    t = label_ref[0]
    H, W = praw.shape
    N = H * W
    _, is_pos, focal = _focal_map(praw, t)
    n_pos_i = _rsum(t).astype(jnp.int32)
    n_neg_i = N - n_pos_i
    k = jnp.minimum(n_pos_i * _HNM, n_neg_i)

    vals = jnp.where(is_pos, -1.0, focal)
    vbits = lax.bitcast_convert_type(vals, jnp.int32)

    def step(_, lh):
        lo, hi = lh
        mid = lo + (hi - lo + 1) // 2
        cnt = jnp.sum(jnp.sum((vbits >= mid).astype(jnp.int32), axis=0))
        take = cnt >= k
        return (jnp.where(take, mid, lo), jnp.where(take, hi, mid - 1))

    lo, _ = lax.fori_loop(0, 31, step, (jnp.int32(0), jnp.int32(0x7F7FFFFF)))
    gt = vbits > lo
    eq = vbits == lo
    cnt_gt = _rsum(gt.astype(jnp.float32))
    cnt_eq = jnp.maximum(_rsum(eq.astype(jnp.float32)), 1.0)
    sum_gt = _rsum(jnp.where(gt, focal, 0.0))
    sum_eq = _rsum(jnp.where(eq, focal, 0.0))
    out_ref[0, 0, 0] = (sum_gt
                        + (k.astype(jnp.float32) - cnt_gt) * sum_eq / cnt_eq)
    out_ref[0, 0, 1] = _rsum(focal * t)


def kernel(prob_map, rain_logit, pred_phys, label_map, rain_max_true,
           rain_spatial_true, phys_targets, phys_mu, phys_std):
    B, H, W = prob_map.shape
    N = H * W
    P = pred_phys.shape[1]
    mu_b = jnp.broadcast_to(phys_mu[None, :], (B, P))
    std_b = jnp.broadcast_to(phys_std[None, :], (B, P))

    img = pl.BlockSpec((1, H, W), lambda b: (b, 0, 0))
    small = pl.BlockSpec((B, P), lambda b: (0, 0))
    stats = pl.pallas_call(
        _body,
        grid=(B,),
        in_specs=[img, img, img, img, small, small, small, small],
        out_specs=pl.BlockSpec((1, 1, 8), lambda b: (b, 0, 0),
                               memory_space=pltpu.SMEM),
        out_shape=jax.ShapeDtypeStruct((B, 1, 8), jnp.float32),
        compiler_params=pltpu.CompilerParams(
            dimension_semantics=("parallel",)),
    )(prob_map, label_map, rain_logit, rain_spatial_true,
      pred_phys, phys_targets, mu_b, std_b)

    stats = stats[:, 0, :]
    sum_focal = stats[:, 0]
    tv_b = stats[:, 1]
    n_pos_f = stats[:, 5]

    n_pos_i = n_pos_f.astype(jnp.int32)
    n_neg_i = N - n_pos_i
    n_hard_i = jnp.minimum(n_pos_i * _HNM, n_neg_i)
    common = n_hard_i == n_neg_i

    def _fallback():
        fb = pl.pallas_call(
            _fb_body,
            grid=(B,),
            in_specs=[img, img],
            out_specs=pl.BlockSpec((1, 1, 8), lambda b: (b, 0, 0),
                                   memory_space=pltpu.SMEM),
            out_shape=jax.ShapeDtypeStruct((B, 1, 8), jnp.float32),
        )(prob_map, label_map)
        return jnp.where(common, sum_focal, fb[:, 0, 1] + fb[:, 0, 0])

    numer = lax.cond(jnp.all(common), lambda: sum_focal, _fallback)

    fl = jnp.mean(numer / (n_pos_f + n_hard_i.astype(jnp.float32)))
    tv = jnp.mean(tv_b)
    reg = jnp.sum(stats[:, 2]) / jnp.maximum(jnp.sum(stats[:, 3]), 1.0)
    aux = stats[0, 4]
    total = fl + 0.5 * tv + 1.0 * reg + 0.1 * aux
    return (total, fl, tv, reg, aux)
